# Initial kernel scaffold; baseline (speedup 1.0000x reference)
import jax
import jax.numpy as jnp
from jax.experimental import pallas as pl


def kernel(inputs, edge_index, W1, b1, W2, b2):
    raise NotImplementedError("write your pallas kernel here")


# speedup vs baseline: 11.4211x; 11.4211x over previous
import functools

import jax
import jax.numpy as jnp
from jax import lax
from jax.experimental import pallas as pl
from jax.experimental.pallas import tpu as pltpu
from jax.experimental.pallas import tpu_sc as plsc

N_NODES = 10000
N_EDGES = 320000
D_IN = 128
D_HID = 128
D_OUT = 16

_NC = 2
_NS = 16
_NW = _NC * _NS
_EPW = N_EDGES // _NW
_B = 125
_NCH = _EPW // _B
_RPT = 624
_TAIL = N_NODES - _RPT * _NS
_ZB = 104
_LANES = 16


def _mesh():
    return plsc.VectorSubcoreMesh(core_axis_name="c", subcore_axis_name="s")


def _deg_body(src_hbm, dst_hbm, deg_src_hbm, deg_dst_hbm, idx_v, cnt_s, cnt_d):
    cid = lax.axis_index("c")
    sid = lax.axis_index("s")
    wid = sid * _NC + cid
    zeros = jnp.zeros((_LANES,), jnp.float32)
    ones = jnp.ones((_LANES,), jnp.float32)

    def zero_body(i, _):
        cnt_s[pl.ds(i * _LANES, _LANES)] = zeros
        cnt_d[pl.ds(i * _LANES, _LANES)] = zeros
        return 0

    lax.fori_loop(0, N_NODES // _LANES, zero_body, 0)

    def _count(idx_hbm, cnt):
        pltpu.sync_copy(idx_hbm.at[pl.ds(wid * _EPW, _EPW)], idx_v)

        def body(i, _):
            iv = idx_v[pl.ds(i * _LANES, _LANES)]
            plsc.addupdate_scatter(cnt, [iv], ones)
            return 0

        lax.fori_loop(0, _EPW // _LANES, body, 0)

    _count(src_hbm, cnt_s)
    _count(dst_hbm, cnt_d)
    pltpu.sync_copy(cnt_s, deg_src_hbm.at[wid])
    pltpu.sync_copy(cnt_d, deg_dst_hbm.at[wid])


_deg = functools.partial(
    pl.kernel,
    out_type=[
        jax.ShapeDtypeStruct((_NW, N_NODES), jnp.float32),
        jax.ShapeDtypeStruct((_NW, N_NODES), jnp.float32),
    ],
    mesh=_mesh(),
    scratch_types=[
        pltpu.VMEM((_EPW,), jnp.int32),
        pltpu.VMEM((N_NODES,), jnp.float32),
        pltpu.VMEM((N_NODES,), jnp.float32),
    ],
    compiler_params=pltpu.CompilerParams(needs_layout_passes=False),
)(_deg_body)


def _make_agg(d):
    @functools.partial(
        pl.kernel,
        out_type=jax.ShapeDtypeStruct((_NC, N_NODES, d), jnp.float32),
        mesh=_mesh(),
        scratch_types=[
            pltpu.VMEM((_NCH, _B), jnp.int32),
            pltpu.VMEM((_NCH, _B), jnp.int32),
            pltpu.VMEM((_B, d), jnp.float32),
            pltpu.VMEM((_ZB, d), jnp.float32),
            pltpu.VMEM_SHARED((N_NODES, d), jnp.float32),
            pltpu.SemaphoreType.DMA,
        ],
        compiler_params=pltpu.CompilerParams(
            use_tc_tiling_on_sc=(d % 128 == 0),
        ),
    )
    def agg(h_hbm, src_hbm, dst_hbm, out_hbm, sidx, didx, buf, zbuf, shared, sem):
        cid = lax.axis_index("c")
        sid = lax.axis_index("s")
        wid = sid * _NC + cid
        pltpu.sync_copy(src_hbm.at[wid], sidx)
        pltpu.sync_copy(dst_hbm.at[wid], didx)

        zeros = jnp.zeros((_LANES,), jnp.float32)

        def zb(i, _):
            for c in range(d // _LANES):
                zbuf[i, pl.ds(c * _LANES, _LANES)] = zeros
            return 0

        lax.fori_loop(0, _ZB, zb, 0)
        base = sid * _RPT
        for r in range(_RPT // _ZB):
            pltpu.sync_copy(zbuf, shared.at[pl.ds(base + r * _ZB, _ZB)])

        @pl.when(sid == _NS - 1)
        def _zero_tail():
            pltpu.sync_copy(
                zbuf.at[pl.ds(0, _TAIL)],
                shared.at[pl.ds(_RPT * _NS, _TAIL)],
            )

        plsc.subcore_barrier()

        def body(j, _):
            pltpu.async_copy(h_hbm.at[sidx.at[j]], buf, sem).wait()
            pltpu.sync_copy(buf, shared.at[didx.at[j]], add=True)
            return 0

        lax.fori_loop(0, _NCH, body, 0)
        plsc.subcore_barrier()
        pltpu.sync_copy(
            shared.at[pl.ds(base, _RPT)],
            out_hbm.at[cid, pl.ds(base, _RPT)],
        )

        @pl.when(sid == _NS - 1)
        def _write_tail():
            pltpu.sync_copy(
                shared.at[pl.ds(_RPT * _NS, _TAIL)],
                out_hbm.at[cid, pl.ds(_RPT * _NS, _TAIL)],
            )

    return agg


_agg_hid = _make_agg(D_HID)
_agg_out = _make_agg(D_OUT)


def _tc1(deg_src_p, deg_dst_p, x, w1):
    def body(ds_ref, dd_ref, x_ref, w_ref, h_ref, ns_ref, nd_ref):
        deg_out = jnp.sum(ds_ref[...], axis=0)
        deg_in = jnp.sum(dd_ref[...], axis=0)
        ns = jnp.where(deg_out > 0, lax.rsqrt(jnp.maximum(deg_out, 1.0)), 0.0)
        nd = jnp.where(deg_in > 0, lax.rsqrt(jnp.maximum(deg_in, 1.0)), 0.0)
        ns_ref[...] = ns
        nd_ref[...] = nd
        h_ref[...] = jnp.dot(
            x_ref[...] * ns[:, None],
            w_ref[...],
            preferred_element_type=jnp.float32,
            precision=lax.Precision.HIGHEST,
        )

    return pl.pallas_call(
        body,
        out_shape=[
            jax.ShapeDtypeStruct((N_NODES, D_HID), jnp.float32),
            jax.ShapeDtypeStruct((N_NODES,), jnp.float32),
            jax.ShapeDtypeStruct((N_NODES,), jnp.float32),
        ],
    )(deg_src_p, deg_dst_p, x, w1)


def _tc2(p1, ndst, b1, nsrc, w2):
    def body(p_ref, nd_ref, b_ref, ns_ref, w_ref, o_ref):
        agg = (p_ref[0] + p_ref[1]) * nd_ref[...][:, None] + b_ref[...][None, :]
        h = jnp.maximum(agg, 0.0)
        o_ref[...] = jnp.dot(
            h * ns_ref[...][:, None],
            w_ref[...],
            preferred_element_type=jnp.float32,
            precision=lax.Precision.HIGHEST,
        )

    return pl.pallas_call(
        body,
        out_shape=jax.ShapeDtypeStruct((N_NODES, D_OUT), jnp.float32),
    )(p1, ndst, b1, nsrc, w2)


def _tc3(p2, ndst, b2):
    def body(p_ref, nd_ref, b_ref, o_ref):
        agg = (p_ref[0] + p_ref[1]) * nd_ref[...][:, None] + b_ref[...][None, :]
        o_ref[...] = jax.nn.sigmoid(agg)

    return pl.pallas_call(
        body,
        out_shape=jax.ShapeDtypeStruct((N_NODES, D_OUT), jnp.float32),
    )(p2, ndst, b2)


def kernel(inputs, edge_index, W1, b1, W2, b2):
    src = edge_index[0].astype(jnp.int32)
    dst = edge_index[1].astype(jnp.int32)
    src3 = src.reshape(_NW, _NCH, _B)
    dst3 = dst.reshape(_NW, _NCH, _B)

    deg_src_p, deg_dst_p = _deg(src, dst)
    h1, nsrc, ndst = _tc1(deg_src_p, deg_dst_p, inputs, W1)
    p1 = _agg_hid(h1, src3, dst3)
    h2 = _tc2(p1, ndst, b1, nsrc, W2)
    p2 = _agg_out(h2, src3, dst3)
    return _tc3(p2, ndst, b2)


# speedup vs baseline: 16.9474x; 1.4839x over previous
import functools

import jax
import jax.numpy as jnp
from jax import lax
from jax.experimental import pallas as pl
from jax.experimental.pallas import tpu as pltpu
from jax.experimental.pallas import tpu_sc as plsc

N_NODES = 10000
N_EDGES = 320000
D_IN = 128
D_HID = 128
D_OUT = 16

_NC = 2
_NS = 16
_NW = _NC * _NS
_EPW = N_EDGES // _NW
_B = 100
_NCH = _EPW // _B
_RPT = 624
_TAIL = N_NODES - _RPT * _NS
_ZCH = 96
_LANES = 16


def _mesh():
    return plsc.VectorSubcoreMesh(core_axis_name="c", subcore_axis_name="s")


def _deg_body(src_hbm, dst_hbm, deg_src_hbm, deg_dst_hbm, idx_v, cnt_s, cnt_d):
    cid = lax.axis_index("c")
    sid = lax.axis_index("s")
    wid = sid * _NC + cid
    zeros = jnp.zeros((_LANES,), jnp.float32)
    ones = jnp.ones((_LANES,), jnp.float32)

    def zero_body(i, _):
        cnt_s[pl.ds(i * _LANES, _LANES)] = zeros
        cnt_d[pl.ds(i * _LANES, _LANES)] = zeros
        return 0

    lax.fori_loop(0, N_NODES // _LANES, zero_body, 0)

    def _count(idx_hbm, cnt):
        pltpu.sync_copy(idx_hbm.at[pl.ds(wid * _EPW, _EPW)], idx_v)

        def body(i, _):
            iv = idx_v[pl.ds(i * _LANES, _LANES)]
            plsc.addupdate_scatter(cnt, [iv], ones)
            return 0

        lax.fori_loop(0, _EPW // _LANES, body, 0)

    _count(src_hbm, cnt_s)
    _count(dst_hbm, cnt_d)
    pltpu.sync_copy(cnt_s, deg_src_hbm.at[wid])
    pltpu.sync_copy(cnt_d, deg_dst_hbm.at[wid])


_deg = functools.partial(
    pl.kernel,
    out_type=[
        jax.ShapeDtypeStruct((_NW, N_NODES), jnp.float32),
        jax.ShapeDtypeStruct((_NW, N_NODES), jnp.float32),
    ],
    mesh=_mesh(),
    scratch_types=[
        pltpu.VMEM((_EPW,), jnp.int32),
        pltpu.VMEM((N_NODES,), jnp.float32),
        pltpu.VMEM((N_NODES,), jnp.float32),
    ],
    compiler_params=pltpu.CompilerParams(needs_layout_passes=False),
)(_deg_body)


def _make_agg(d, nbuf):
    assert _NCH % nbuf == 0

    @functools.partial(
        pl.kernel,
        out_type=jax.ShapeDtypeStruct((_NC, N_NODES, d), jnp.float32),
        mesh=_mesh(),
        scratch_types=[
            pltpu.VMEM((_NCH, _B), jnp.int32),
            pltpu.VMEM((_NCH, _B), jnp.int32),
            pltpu.VMEM_SHARED((N_NODES, d), jnp.float32),
        ]
        + [pltpu.VMEM((_B, d), jnp.float32) for _ in range(nbuf)]
        + [pltpu.SemaphoreType.DMA for _ in range(nbuf)],
        compiler_params=pltpu.CompilerParams(
            use_tc_tiling_on_sc=False,
        ),
    )
    def agg(h_hbm, src_hbm, dst_hbm, out_hbm, sidx, didx, shared, *rest):
        bufs = rest[:nbuf]
        sems = rest[nbuf:]
        cid = lax.axis_index("c")
        sid = lax.axis_index("s")
        wid = sid * _NC + cid
        pltpu.sync_copy(src_hbm.at[wid], sidx)
        pltpu.sync_copy(dst_hbm.at[wid], didx)

        zeros = jnp.zeros((_LANES,), jnp.float32)

        def zb(i, _):
            for c in range(d // _LANES):
                bufs[0][i, pl.ds(c * _LANES, _LANES)] = zeros
            return 0

        lax.fori_loop(0, _B, zb, 0)
        base = sid * _RPT
        for r in range(_RPT // _ZCH):
            pltpu.sync_copy(
                bufs[0].at[pl.ds(0, _ZCH)],
                shared.at[pl.ds(base + r * _ZCH, _ZCH)],
            )
        pltpu.sync_copy(
            bufs[0].at[pl.ds(0, _RPT % _ZCH)],
            shared.at[pl.ds(base + (_RPT // _ZCH) * _ZCH, _RPT % _ZCH)],
        )

        @pl.when(sid == _NS - 1)
        def _zero_tail():
            pltpu.sync_copy(
                bufs[0].at[pl.ds(0, _TAIL)],
                shared.at[pl.ds(_RPT * _NS, _TAIL)],
            )

        plsc.subcore_barrier()

        for b in range(nbuf):
            pltpu.async_copy(h_hbm.at[sidx.at[b]], bufs[b], sems[b])

        def body(i, _):
            j0 = i * nbuf
            for b in range(nbuf):
                j = j0 + b
                pltpu.make_async_copy(
                    h_hbm.at[sidx.at[j]], bufs[b], sems[b]
                ).wait()
                pltpu.sync_copy(bufs[b], shared.at[didx.at[j]], add=True)

                @pl.when(j + nbuf < _NCH)
                def _prefetch():
                    pltpu.async_copy(
                        h_hbm.at[sidx.at[j + nbuf]], bufs[b], sems[b]
                    )

            return 0

        lax.fori_loop(0, _NCH // nbuf, body, 0)
        plsc.subcore_barrier()
        pltpu.sync_copy(
            shared.at[pl.ds(base, _RPT)],
            out_hbm.at[cid, pl.ds(base, _RPT)],
        )

        @pl.when(sid == _NS - 1)
        def _write_tail():
            pltpu.sync_copy(
                shared.at[pl.ds(_RPT * _NS, _TAIL)],
                out_hbm.at[cid, pl.ds(_RPT * _NS, _TAIL)],
            )

    return agg


_agg_hid = _make_agg(D_HID, 2)
_agg_out = _make_agg(D_OUT, 4)


def _tc1(deg_src_p, deg_dst_p, x, w1):
    def body(ds_ref, dd_ref, x_ref, w_ref, h_ref, ns_ref, nd_ref):
        deg_out = jnp.sum(ds_ref[...], axis=0)
        deg_in = jnp.sum(dd_ref[...], axis=0)
        ns = jnp.where(deg_out > 0, lax.rsqrt(jnp.maximum(deg_out, 1.0)), 0.0)
        nd = jnp.where(deg_in > 0, lax.rsqrt(jnp.maximum(deg_in, 1.0)), 0.0)
        ns_ref[...] = ns
        nd_ref[...] = nd
        h_ref[...] = jnp.dot(
            x_ref[...] * ns[:, None],
            w_ref[...],
            preferred_element_type=jnp.float32,
            precision=lax.Precision.HIGHEST,
        )

    return pl.pallas_call(
        body,
        out_shape=[
            jax.ShapeDtypeStruct((N_NODES, D_HID), jnp.float32),
            jax.ShapeDtypeStruct((N_NODES,), jnp.float32),
            jax.ShapeDtypeStruct((N_NODES,), jnp.float32),
        ],
    )(deg_src_p, deg_dst_p, x, w1)


def _tc2(p1, ndst, b1, nsrc, w2):
    def body(p_ref, nd_ref, b_ref, ns_ref, w_ref, o_ref):
        agg = (p_ref[0] + p_ref[1]) * nd_ref[...][:, None] + b_ref[...][None, :]
        h = jnp.maximum(agg, 0.0)
        o_ref[...] = jnp.dot(
            h * ns_ref[...][:, None],
            w_ref[...],
            preferred_element_type=jnp.float32,
            precision=lax.Precision.HIGHEST,
        )

    return pl.pallas_call(
        body,
        out_shape=jax.ShapeDtypeStruct((N_NODES, D_OUT), jnp.float32),
    )(p1, ndst, b1, nsrc, w2)


def _tc3(p2, ndst, b2):
    def body(p_ref, nd_ref, b_ref, o_ref):
        agg = (p_ref[0] + p_ref[1]) * nd_ref[...][:, None] + b_ref[...][None, :]
        o_ref[...] = jax.nn.sigmoid(agg)

    return pl.pallas_call(
        body,
        out_shape=jax.ShapeDtypeStruct((N_NODES, D_OUT), jnp.float32),
    )(p2, ndst, b2)


def kernel(inputs, edge_index, W1, b1, W2, b2):
    src = edge_index[0].astype(jnp.int32)
    dst = edge_index[1].astype(jnp.int32)
    src3 = src.reshape(_NW, _NCH, _B)
    dst3 = dst.reshape(_NW, _NCH, _B)

    deg_src_p, deg_dst_p = _deg(src, dst)
    h1, nsrc, ndst = _tc1(deg_src_p, deg_dst_p, inputs, W1)
    p1 = _agg_hid(h1, src3, dst3)
    h2 = _tc2(p1, ndst, b1, nsrc, W2)
    p2 = _agg_out(h2, src3, dst3)
    return _tc3(p2, ndst, b2)
